# baseline (speedup 1.0000x reference)
import numpy as np
import jax
import jax.numpy as jnp
from jax import lax
from jax.experimental import pallas as pl
from jax.experimental.pallas import tpu as pltpu

ALPHA = 9.0
FG = 0.5
BG = 0.4
RATIOS = [0.5, 1.0, 2.0]
SCALES = [2.0, 4.0, 8.0]
STRIDE = 16
NUM_CLS = 2
NUM_REG = 12
A = 9
PADA = 16
H, W = 36, 120
P = H * W


def _geometry():
    shapes = []
    for s in SCALES:
        for r in RATIOS:
            size = STRIDE * s
            shapes.append((size * np.sqrt(r), size / np.sqrt(r)))
    shapes = np.array(shapes, dtype=np.float32)
    halves = shapes / 2.0
    wa = np.zeros((PADA, 1), np.float32)
    ha = np.zeros((PADA, 1), np.float32)
    wa[:A, 0] = halves[:, 0]
    ha[:A, 0] = halves[:, 1]
    cy = (np.arange(H, dtype=np.float32) + 0.5) * STRIDE
    cx = (np.arange(W, dtype=np.float32) + 0.5) * STRIDE
    cxp = np.tile(cx, H)[None, :]
    cyp = np.repeat(cy, W)[None, :]
    ax1 = cxp - wa
    ax2 = cxp + wa
    ay1 = cyp - ha
    ay2 = cyp + ha
    area = (ax2 - ax1) * (ay2 - ay1)
    return np.concatenate([ax1, ax2, ay1, ay2, area], axis=0)


_GEOM = _geometry()

_NB = NUM_CLS + NUM_REG
_PC = np.zeros((_NB * PADA, NUM_CLS * A), np.float32)
_PR = np.zeros((_NB * PADA, NUM_REG * A), np.float32)
for _k in range(_NB):
    for _a in range(A):
        if _k < NUM_CLS:
            _PC[_k * PADA + _a, _a * NUM_CLS + _k] = 1.0
        else:
            _PR[_k * PADA + _a, _a * NUM_REG + (_k - NUM_CLS)] = 1.0


def _loss_kernel(f_ref, wc_ref, wr_ref, bc_ref, br_ref, pc_ref, pr_ref,
                 g_ref, ann_ref, cls_out, reg_out, acc_ref):
    bi = pl.program_id(0)
    nb = pl.num_programs(0)

    pc = pc_ref[...]
    pr = pr_ref[...]
    w_all = (jnp.dot(pc, wc_ref[...], preferred_element_type=jnp.float32)
             + jnp.dot(pr, wr_ref[...], preferred_element_type=jnp.float32))
    b_all = (jnp.dot(pc, bc_ref[...], preferred_element_type=jnp.float32)
             + jnp.dot(pr, br_ref[...], preferred_element_type=jnp.float32))

    f = f_ref[0].reshape(f_ref.shape[1], P)
    logits = jnp.dot(w_all, f, preferred_element_type=jnp.float32) + b_all

    ax1 = g_ref[0:PADA]
    ax2 = g_ref[PADA:2 * PADA]
    ay1 = g_ref[2 * PADA:3 * PADA]
    ay2 = g_ref[3 * PADA:4 * PADA]
    area_a = g_ref[4 * PADA:5 * PADA]

    best_i = jnp.full((PADA, P), -1.0, jnp.float32)
    best_u = jnp.ones((PADA, P), jnp.float32)
    tr = [jnp.zeros((PADA, P), jnp.float32) for _ in range(NUM_REG)]
    for m in range(8):
        bx1 = ann_ref[bi, m, 4]
        by1 = ann_ref[bi, m, 5]
        bx2 = ann_ref[bi, m, 6]
        by2 = ann_ref[bi, m, 7]
        area_b = (bx2 - bx1) * (by2 - by1)
        iw = jnp.maximum(jnp.minimum(ax2, bx2) - jnp.maximum(ax1, bx1), 0.0)
        ih = jnp.maximum(jnp.minimum(ay2, by2) - jnp.maximum(ay1, by1), 0.0)
        inter = iw * ih
        union = area_a + area_b - inter
        upd = inter * best_u > best_i * union
        best_i = jnp.where(upd, inter, best_i)
        best_u = jnp.where(upd, union, best_u)
        for r in range(NUM_REG):
            tr[r] = jnp.where(upd, ann_ref[bi, m, r], tr[r])

    rvalid = lax.broadcasted_iota(jnp.int32, (PADA, 1), 0) < A
    pos = (best_i > FG * best_u) & rvalid
    neg = (best_i < BG * best_u) & rvalid
    assigned = pos | neg

    x0 = logits[0:PADA]
    x1 = logits[PADA:2 * PADA]
    ls0 = jax.nn.log_sigmoid(x0)
    ls0m = ls0 - x0
    p0 = jnp.exp(ls0)
    ls1 = jax.nn.log_sigmoid(x1)
    ls1m = ls1 - x1
    p1 = jnp.exp(ls1)
    fb0 = jnp.where(pos, 1.0 - p0, p0)
    cl0 = jnp.where(assigned, fb0 * fb0 * jnp.where(pos, -ls0, -ls0m), 0.0)
    cl0 = jnp.where(cl0 < 1e-5, 0.0, cl0)
    cl1 = jnp.where(assigned, (p1 * p1) * (-ls1m), 0.0)
    cl1 = jnp.where(cl1 < 1e-5, 0.0, cl1)
    cls_part = jnp.sum(cl0) + jnp.sum(cl1)

    reg_part = jnp.float32(0.0)
    for r in range(NUM_REG):
        pred = logits[(2 + r) * PADA:(3 + r) * PADA]
        d = jnp.abs(tr[r] - pred)
        l = jnp.maximum(d - 0.5 / ALPHA,
                        jnp.minimum(0.5 * ALPHA * d * d, 0.5 / ALPHA))
        l = jnp.where(d <= 0.01, 0.0, l)
        reg_part = reg_part + jnp.sum(jnp.where(pos, l, 0.0))

    cnt_part = jnp.sum(jnp.where(pos, 1.0, 0.0))

    @pl.when(bi == 0)
    def _():
        acc_ref[0, 0] = 0.0
        acc_ref[0, 1] = 0.0
        acc_ref[0, 2] = 0.0

    acc_ref[0, 0] += cls_part
    acc_ref[0, 1] += reg_part
    acc_ref[0, 2] += cnt_part

    @pl.when(bi == nb - 1)
    def _():
        cnt = acc_ref[0, 2]
        cls_out[0, 0] = acc_ref[0, 0] / (cnt + 1e-6)
        reg_out[0, 0] = jnp.where(
            cnt > 0.0, acc_ref[0, 1] / jnp.maximum(cnt, 1.0), 0.0)


def kernel(features, P2, annotations, W_cls, b_cls, W_reg, b_reg):
    B, C, Hf, Wf = features.shape

    outs = pl.pallas_call(
        _loss_kernel,
        grid=(B,),
        in_specs=[
            pl.BlockSpec((1, C, Hf, Wf), lambda b: (b, 0, 0, 0)),
            pl.BlockSpec((NUM_CLS * A, C), lambda b: (0, 0)),
            pl.BlockSpec((NUM_REG * A, C), lambda b: (0, 0)),
            pl.BlockSpec((NUM_CLS * A, 1), lambda b: (0, 0)),
            pl.BlockSpec((NUM_REG * A, 1), lambda b: (0, 0)),
            pl.BlockSpec((_NB * PADA, NUM_CLS * A), lambda b: (0, 0)),
            pl.BlockSpec((_NB * PADA, NUM_REG * A), lambda b: (0, 0)),
            pl.BlockSpec((5 * PADA, P), lambda b: (0, 0)),
            pl.BlockSpec((B, 8, NUM_REG), lambda b: (0, 0, 0),
                         memory_space=pltpu.SMEM),
        ],
        out_specs=[
            pl.BlockSpec((1, 1), lambda b: (0, 0), memory_space=pltpu.SMEM),
            pl.BlockSpec((1, 1), lambda b: (0, 0), memory_space=pltpu.SMEM),
        ],
        out_shape=[jax.ShapeDtypeStruct((1, 1), jnp.float32)] * 2,
        scratch_shapes=[pltpu.SMEM((1, 3), jnp.float32)],
        compiler_params=pltpu.CompilerParams(
            dimension_semantics=("arbitrary",)),
    )(features, W_cls, W_reg, b_cls.reshape(-1, 1), b_reg.reshape(-1, 1),
      jnp.asarray(_PC), jnp.asarray(_PR), jnp.asarray(_GEOM), annotations)

    return outs[0].reshape(()), outs[1].reshape(())


# speedup vs baseline: 1.2214x; 1.2214x over previous
import numpy as np
import jax
import jax.numpy as jnp
from jax import lax
from jax.experimental import pallas as pl
from jax.experimental.pallas import tpu as pltpu

ALPHA = 9.0
FG = 0.5
BG = 0.4
RATIOS = [0.5, 1.0, 2.0]
SCALES = [2.0, 4.0, 8.0]
STRIDE = 16
NUM_CLS = 2
NUM_REG = 12
A = 9
PADA = 16
H, W = 36, 120
P = H * W


def _geometry():
    shapes = []
    for s in SCALES:
        for r in RATIOS:
            size = STRIDE * s
            shapes.append((size * np.sqrt(r), size / np.sqrt(r)))
    shapes = np.array(shapes, dtype=np.float32)
    halves = shapes / 2.0
    wa = np.zeros((PADA, 1), np.float32)
    ha = np.zeros((PADA, 1), np.float32)
    wa[:A, 0] = halves[:, 0]
    ha[:A, 0] = halves[:, 1]
    cy = (np.arange(H, dtype=np.float32) + 0.5) * STRIDE
    cx = (np.arange(W, dtype=np.float32) + 0.5) * STRIDE
    cxp = np.tile(cx, H)[None, :]
    cyp = np.repeat(cy, W)[None, :]
    ax1 = cxp - wa
    ax2 = cxp + wa
    ay1 = cyp - ha
    ay2 = cyp + ha
    area = (ax2 - ax1) * (ay2 - ay1)
    return np.concatenate([ax1, ax2, ay1, ay2, area], axis=0)


_GEOM = _geometry()

_NB = NUM_CLS + NUM_REG
_PC = np.zeros((_NB * PADA, NUM_CLS * A), np.float32)
_PR = np.zeros((_NB * PADA, NUM_REG * A), np.float32)
for _k in range(_NB):
    for _a in range(A):
        if _k < NUM_CLS:
            _PC[_k * PADA + _a, _a * NUM_CLS + _k] = 1.0
        else:
            _PR[_k * PADA + _a, _a * NUM_REG + (_k - NUM_CLS)] = 1.0


def _loss_kernel(f_ref, wc_ref, wr_ref, bc_ref, br_ref, pc_ref, pr_ref,
                 g_ref, ann_ref, cls_out, reg_out, acc_ref):
    bi = pl.program_id(0)
    nb = pl.num_programs(0)

    pc = pc_ref[...]
    pr = pr_ref[...]
    w_all = (jnp.dot(pc, wc_ref[...], preferred_element_type=jnp.float32)
             + jnp.dot(pr, wr_ref[...], preferred_element_type=jnp.float32))
    b_all = (jnp.dot(pc, bc_ref[...], preferred_element_type=jnp.float32)
             + jnp.dot(pr, br_ref[...], preferred_element_type=jnp.float32))

    f = f_ref[0]
    logits = jnp.dot(w_all, f, preferred_element_type=jnp.float32) + b_all

    ax1 = g_ref[0:PADA]
    ax2 = g_ref[PADA:2 * PADA]
    ay1 = g_ref[2 * PADA:3 * PADA]
    ay2 = g_ref[3 * PADA:4 * PADA]
    area_a = g_ref[4 * PADA:5 * PADA]

    best_i = jnp.full((PADA, P), -1.0, jnp.float32)
    best_u = jnp.ones((PADA, P), jnp.float32)
    tr = [jnp.zeros((PADA, P), jnp.float32) for _ in range(NUM_REG)]
    for m in range(8):
        bx1 = ann_ref[bi, m, 4]
        by1 = ann_ref[bi, m, 5]
        bx2 = ann_ref[bi, m, 6]
        by2 = ann_ref[bi, m, 7]
        area_b = (bx2 - bx1) * (by2 - by1)
        iw = jnp.maximum(jnp.minimum(ax2, bx2) - jnp.maximum(ax1, bx1), 0.0)
        ih = jnp.maximum(jnp.minimum(ay2, by2) - jnp.maximum(ay1, by1), 0.0)
        inter = iw * ih
        union = area_a + area_b - inter
        upd = inter * best_u > best_i * union
        best_i = jnp.where(upd, inter, best_i)
        best_u = jnp.where(upd, union, best_u)
        for r in range(NUM_REG):
            tr[r] = jnp.where(upd, ann_ref[bi, m, r], tr[r])

    rvalid = lax.broadcasted_iota(jnp.int32, (PADA, 1), 0) < A
    pos = (best_i > FG * best_u) & rvalid
    neg = (best_i < BG * best_u) & rvalid
    assigned = pos | neg

    x0 = logits[0:PADA]
    x1 = logits[PADA:2 * PADA]
    ls0 = jax.nn.log_sigmoid(x0)
    ls0m = ls0 - x0
    p0 = jnp.exp(ls0)
    ls1 = jax.nn.log_sigmoid(x1)
    ls1m = ls1 - x1
    p1 = jnp.exp(ls1)
    fb0 = jnp.where(pos, 1.0 - p0, p0)
    cl0 = jnp.where(assigned, fb0 * fb0 * jnp.where(pos, -ls0, -ls0m), 0.0)
    cl0 = jnp.where(cl0 < 1e-5, 0.0, cl0)
    cl1 = jnp.where(assigned, (p1 * p1) * (-ls1m), 0.0)
    cl1 = jnp.where(cl1 < 1e-5, 0.0, cl1)
    cls_part = jnp.sum(cl0) + jnp.sum(cl1)

    reg_part = jnp.float32(0.0)
    for r in range(NUM_REG):
        pred = logits[(2 + r) * PADA:(3 + r) * PADA]
        d = jnp.abs(tr[r] - pred)
        l = jnp.maximum(d - 0.5 / ALPHA,
                        jnp.minimum(0.5 * ALPHA * d * d, 0.5 / ALPHA))
        l = jnp.where(d <= 0.01, 0.0, l)
        reg_part = reg_part + jnp.sum(jnp.where(pos, l, 0.0))

    cnt_part = jnp.sum(jnp.where(pos, 1.0, 0.0))

    @pl.when(bi == 0)
    def _():
        acc_ref[0, 0] = 0.0
        acc_ref[0, 1] = 0.0
        acc_ref[0, 2] = 0.0

    acc_ref[0, 0] += cls_part
    acc_ref[0, 1] += reg_part
    acc_ref[0, 2] += cnt_part

    @pl.when(bi == nb - 1)
    def _():
        cnt = acc_ref[0, 2]
        cls_out[0, 0] = acc_ref[0, 0] / (cnt + 1e-6)
        reg_out[0, 0] = jnp.where(
            cnt > 0.0, acc_ref[0, 1] / jnp.maximum(cnt, 1.0), 0.0)


def kernel(features, P2, annotations, W_cls, b_cls, W_reg, b_reg):
    B, C, Hf, Wf = features.shape
    f3 = features.reshape(B, C, Hf * Wf)

    outs = pl.pallas_call(
        _loss_kernel,
        grid=(B,),
        in_specs=[
            pl.BlockSpec((1, C, P), lambda b: (b, 0, 0)),
            pl.BlockSpec((NUM_CLS * A, C), lambda b: (0, 0)),
            pl.BlockSpec((NUM_REG * A, C), lambda b: (0, 0)),
            pl.BlockSpec((NUM_CLS * A, 1), lambda b: (0, 0)),
            pl.BlockSpec((NUM_REG * A, 1), lambda b: (0, 0)),
            pl.BlockSpec((_NB * PADA, NUM_CLS * A), lambda b: (0, 0)),
            pl.BlockSpec((_NB * PADA, NUM_REG * A), lambda b: (0, 0)),
            pl.BlockSpec((5 * PADA, P), lambda b: (0, 0)),
            pl.BlockSpec((B, 8, NUM_REG), lambda b: (0, 0, 0),
                         memory_space=pltpu.SMEM),
        ],
        out_specs=[
            pl.BlockSpec((1, 1), lambda b: (0, 0), memory_space=pltpu.SMEM),
            pl.BlockSpec((1, 1), lambda b: (0, 0), memory_space=pltpu.SMEM),
        ],
        out_shape=[jax.ShapeDtypeStruct((1, 1), jnp.float32)] * 2,
        scratch_shapes=[pltpu.SMEM((1, 3), jnp.float32)],
        compiler_params=pltpu.CompilerParams(
            dimension_semantics=("arbitrary",)),
    )(f3, W_cls, W_reg, b_cls.reshape(-1, 1), b_reg.reshape(-1, 1),
      jnp.asarray(_PC), jnp.asarray(_PR), jnp.asarray(_GEOM), annotations)

    return outs[0].reshape(()), outs[1].reshape(())
